# baseline (speedup 1.0000x reference)
import functools

import jax
import jax.numpy as jnp
from jax import lax
from jax.experimental import pallas as pl
from jax.experimental.pallas import tpu as pltpu
from jax.experimental.pallas import tpu_sc as plsc

VOCAB = 1000000
DIM = 64
B = 16384
K = 20

NC = 2
NS = 16
NW = NC * NS
EW = B // NW
CB = 16
NCH = EW // CB
NEG_CH = CB * K
NBUF = 2
NEG_SPLITS = [(s, min(128, NEG_CH - s)) for s in range(0, NEG_CH, 128)]


def _sc_scores_kernel(center, context, neg, in_tab, out_tab,
                      pos_out, neg_out,
                      cen_idx, ctx_idx, neg_idx,
                      cen_off, ctx_off, neg_off,
                      cen_rows, pos_rows, neg_rows,
                      pos_sc, neg_sc, *sems):
    wid = lax.axis_index("s") * NC + lax.axis_index("c")
    base = wid * EW

    pltpu.sync_copy(center.at[pl.ds(base, EW)], cen_idx)
    pltpu.sync_copy(context.at[pl.ds(base, EW)], ctx_idx)
    pltpu.sync_copy(neg.at[pl.ds(base * K, EW * K)], neg_idx)

    iota = lax.iota(jnp.int32, 16)

    def split_cp(i, c):
        for idx_ref, off_ref in ((cen_idx, cen_off), (ctx_idx, ctx_off)):
            v = idx_ref[pl.ds(i * 16, 16)]
            off_ref[pl.ds(i * 16, 16)] = jnp.left_shift(
                jnp.bitwise_and(v, 1), 6)
            idx_ref[pl.ds(i * 16, 16)] = jnp.right_shift(v, 1)
        return c
    lax.fori_loop(0, EW // 16, split_cp, 0)

    stride_iota = iota * K
    for k in range(K):
        def split_neg_off(i, c, k=k):
            v = plsc.load_gather(neg_idx, [stride_iota + (i * (16 * K) + k)])
            neg_off[pl.ds(k * EW + i * 16, 16)] = jnp.left_shift(
                jnp.bitwise_and(v, 1), 6)
            return c
        lax.fori_loop(0, EW // 16, split_neg_off, 0)

    def split_neg(i, c):
        v = neg_idx[pl.ds(i * 16, 16)]
        neg_idx[pl.ds(i * 16, 16)] = jnp.right_shift(v, 1)
        return c
    lax.fori_loop(0, (EW * K) // 16, split_neg, 0)

    def fire(j, par):
        descs = [
            pltpu.make_async_copy(
                in_tab.at[cen_idx.at[pl.ds(j * CB, CB)]],
                cen_rows.at[par], sems[par]),
            pltpu.make_async_copy(
                out_tab.at[ctx_idx.at[pl.ds(j * CB, CB)]],
                pos_rows.at[par], sems[par]),
        ]
        for s, n in NEG_SPLITS:
            descs.append(pltpu.make_async_copy(
                out_tab.at[neg_idx.at[pl.ds(j * NEG_CH + s, n)]],
                neg_rows.at[par, pl.ds(s, n)], sems[par]))
        for d in descs:
            d.start()
        return descs

    def wait(j, par):
        del j
        pltpu.make_async_copy(
            in_tab.at[cen_idx.at[pl.ds(0, CB)]], cen_rows.at[par],
            sems[par]).wait()
        pltpu.make_async_copy(
            out_tab.at[ctx_idx.at[pl.ds(0, CB)]], pos_rows.at[par],
            sems[par]).wait()
        for s, n in NEG_SPLITS:
            pltpu.make_async_copy(
                out_tab.at[neg_idx.at[pl.ds(s, n)]],
                neg_rows.at[par, pl.ds(s, n)], sems[par]).wait()

    rows128 = iota * 128
    zero16 = jnp.zeros((16,), jnp.int32)

    def compute(j, par):
        crows = cen_rows.at[par]
        prows = pos_rows.at[par]
        nrows = neg_rows.at[par]
        cbase = rows128 + cen_off[pl.ds(j * CB, 16)]
        pbase = rows128 + ctx_off[pl.ds(j * CB, 16)]
        nbases = [iota * (K * 128) + k * 128 +
                  neg_off[pl.ds(k * EW + j * CB, 16)] for k in range(K)]

        def body(d, carry):
            dcol = carry[0]
            cd = plsc.load_gather(crows, [zero16, cbase + dcol])
            pd = plsc.load_gather(prows, [zero16, pbase + dcol])
            accs = [dcol + 1, carry[1] + cd * pd]
            for k in range(K):
                nd = plsc.load_gather(nrows, [zero16, nbases[k] + dcol])
                accs.append(carry[k + 2] + cd * nd)
            return tuple(accs)

        init = (zero16,) + tuple(
            jnp.zeros((16,), jnp.float32) for _ in range(K + 1))
        accs = lax.fori_loop(0, DIM, body, init)

        pos_sc[pl.ds(j * CB, 16)] = accs[1]
        for k in range(K):
            neg_sc[pl.ds(k * EW + j * CB, 16)] = accs[k + 2]

    fire(0, 0)
    fire(1, 1)

    def chunk_pair(t, c):
        j = t * 2

        @pl.when(j + 2 < NCH)
        def _():
            fire(j + 2, 0)
        wait(j, 0)
        compute(j, 0)

        @pl.when(j + 3 < NCH)
        def _():
            fire(j + 3, 1)
        wait(j + 1, 1)
        compute(j + 1, 1)
        return c

    lax.fori_loop(0, NCH // 2, chunk_pair, 0)

    pltpu.sync_copy(pos_sc, pos_out.at[pl.ds(base, EW)])
    pltpu.sync_copy(neg_sc, neg_out.at[pl.ds(base * K, EW * K)])


@jax.jit
def _sc_scores(center, context, neg_flat, in_tab2, out_tab2):
    mesh = plsc.VectorSubcoreMesh(core_axis_name="c", subcore_axis_name="s")
    return pl.kernel(
        _sc_scores_kernel,
        out_type=(
            jax.ShapeDtypeStruct((B,), jnp.float32),
            jax.ShapeDtypeStruct((B * K,), jnp.float32),
        ),
        mesh=mesh,
        scratch_types=(
            pltpu.VMEM((EW,), jnp.int32),
            pltpu.VMEM((EW,), jnp.int32),
            pltpu.VMEM((EW * K,), jnp.int32),
            pltpu.VMEM((EW,), jnp.int32),
            pltpu.VMEM((EW,), jnp.int32),
            pltpu.VMEM((K * EW,), jnp.int32),
            pltpu.VMEM((NBUF, CB, 128), jnp.float32),
            pltpu.VMEM((NBUF, CB, 128), jnp.float32),
            pltpu.VMEM((NBUF, NEG_CH, 128), jnp.float32),
            pltpu.VMEM((EW,), jnp.float32),
            pltpu.VMEM((K * EW,), jnp.float32),
        ) + (pltpu.SemaphoreType.DMA,) * NBUF,
        compiler_params=pltpu.CompilerParams(needs_layout_passes=False,
                                             disable_bounds_checks=True),
        name="w2v_sc_scores",
    )(center, context, neg_flat, in_tab2, out_tab2)


def _tc_loss_kernel(pos_ref, neg_ref, out_ref):
    def ls(x):
        return jnp.minimum(x, 0.0) - jnp.log1p(jnp.exp(-jnp.abs(x)))

    total = jnp.sum(ls(pos_ref[...])) + jnp.sum(ls(-neg_ref[...]))
    out_ref[...] = jnp.full((1, 1), -1.0 / B) * total


def _tc_loss(pos, negs):
    pos2 = pos.reshape(128, 128)
    neg2 = negs.reshape(B * K // 128, 128)
    out = pl.pallas_call(
        _tc_loss_kernel,
        out_shape=jax.ShapeDtypeStruct((1, 1), jnp.float32),
    )(pos2, neg2)
    return out.reshape(())


def kernel(center, context, negatives, input_table, output_table):
    neg_flat = negatives.reshape(B * K)
    in_tab2 = input_table.reshape(VOCAB // 2, 2 * DIM)
    out_tab2 = output_table.reshape(VOCAB // 2, 2 * DIM)
    pos_sc, neg_sc = _sc_scores(center.astype(jnp.int32),
                                context.astype(jnp.int32),
                                neg_flat.astype(jnp.int32),
                                in_tab2, out_tab2)
    return _tc_loss(pos_sc, neg_sc)
